# Initial kernel scaffold; baseline (speedup 1.0000x reference)
import jax
import jax.numpy as jnp
from jax.experimental import pallas as pl


def kernel(z, elec, W_elec, nuclare_table, W_ls, b_ls):
    raise NotImplementedError("write your pallas kernel here")


# speedup vs baseline: 2.1084x; 2.1084x over previous
import functools

import jax
import jax.numpy as jnp
from jax import lax
from jax.experimental import pallas as pl
from jax.experimental.pallas import tpu as pltpu
from jax.experimental.pallas import tpu_sc as plsc

NUM_FEATURES = 128
VOCAB = 37
N_ATOMS = 100000

_NC = 2
_NS = 16
_NW = _NC * _NS

_CHUNK = 128
_NCHUNK_PAD = 800
_N_FULL = N_ATOMS // _CHUNK
_REM = N_ATOMS - _N_FULL * _CHUNK
_CHUNKS_PER_W = _NCHUNK_PAD // _NW


def _table_body(elec_ref, we_ref, nuc_ref, wls_ref, b_ref, out_ref):
    h = nuc_ref[...] + lax.dot_general(
        elec_ref[...], we_ref[...], (((1,), (1,)), ((), ())),
        preferred_element_type=jnp.float32)
    o = lax.dot_general(
        h, wls_ref[...], (((1,), (1,)), ((), ())),
        preferred_element_type=jnp.float32) + b_ref[...]
    out_ref[...] = o * jax.nn.sigmoid(o)


def _compute_table(elec, W_elec, nuclare_table, W_ls, b_ls):
    return pl.pallas_call(
        _table_body,
        out_shape=jax.ShapeDtypeStruct((VOCAB, NUM_FEATURES), jnp.float32),
    )(elec, W_elec, nuclare_table, W_ls, b_ls.reshape(1, NUM_FEATURES))


_mesh = plsc.VectorSubcoreMesh(core_axis_name="c", subcore_axis_name="s")


@functools.partial(
    pl.kernel,
    mesh=_mesh,
    out_type=jax.ShapeDtypeStruct((N_ATOMS, NUM_FEATURES), jnp.float32),
    scratch_types=[
        pltpu.VMEM((_CHUNK,), jnp.int32),
        pltpu.VMEM((_CHUNK, NUM_FEATURES), jnp.float32),
        pltpu.SemaphoreType.DMA,
    ],
)
def _sc_gather(table_hbm, z2d_hbm, out_hbm, idx_v, rows_v, sem):
    wid = lax.axis_index("s") * _NC + lax.axis_index("c")

    def body(j, carry):
        g = wid * _CHUNKS_PER_W + j

        @pl.when(g <= _N_FULL)
        def _():
            pltpu.sync_copy(z2d_hbm.at[g], idx_v)
            pltpu.async_copy(table_hbm.at[idx_v], rows_v, sem).wait()

            @pl.when(g < _N_FULL)
            def _():
                pltpu.sync_copy(rows_v, out_hbm.at[pl.ds(g * _CHUNK, _CHUNK)])

            @pl.when(g == _N_FULL)
            def _():
                pltpu.sync_copy(
                    rows_v.at[pl.ds(0, _REM)],
                    out_hbm.at[pl.ds(_N_FULL * _CHUNK, _REM)])

        return carry

    lax.fori_loop(0, _CHUNKS_PER_W, body, 0)


def kernel(z, elec, W_elec, nuclare_table, W_ls, b_ls):
    table = _compute_table(elec, W_elec, nuclare_table, W_ls, b_ls)
    z_pad = jnp.pad(z, (0, _NCHUNK_PAD * _CHUNK - N_ATOMS))
    z2d = z_pad.reshape(_NCHUNK_PAD, _CHUNK)
    return _sc_gather(table, z2d)


# speedup vs baseline: 2.1930x; 1.0402x over previous
import functools

import jax
import jax.numpy as jnp
from jax import lax
from jax.experimental import pallas as pl
from jax.experimental.pallas import tpu as pltpu
from jax.experimental.pallas import tpu_sc as plsc

NUM_FEATURES = 128
VOCAB = 37
N_ATOMS = 100000

_NC = 2
_NS = 16
_NW = _NC * _NS

_CHUNK = 128
_NCHUNKS = 800
_N_FULL = N_ATOMS // _CHUNK
_REM = N_ATOMS - _N_FULL * _CHUNK
_JPW = _NCHUNKS // _NW
_NBUF = 4


def _table_body(elec_ref, we_ref, nuc_ref, wls_ref, b_ref, out_ref):
    h = nuc_ref[...] + lax.dot_general(
        elec_ref[...], we_ref[...], (((1,), (1,)), ((), ())),
        preferred_element_type=jnp.float32)
    o = lax.dot_general(
        h, wls_ref[...], (((1,), (1,)), ((), ())),
        preferred_element_type=jnp.float32) + b_ref[...]
    out_ref[...] = o * jax.nn.sigmoid(o)


def _compute_table(elec, W_elec, nuclare_table, W_ls, b_ls):
    return pl.pallas_call(
        _table_body,
        out_shape=jax.ShapeDtypeStruct((VOCAB, NUM_FEATURES), jnp.float32),
    )(elec, W_elec, nuclare_table, W_ls, b_ls.reshape(1, NUM_FEATURES))


_mesh = plsc.VectorSubcoreMesh(core_axis_name="c", subcore_axis_name="s")


@functools.partial(
    pl.kernel,
    mesh=_mesh,
    out_type=jax.ShapeDtypeStruct((N_ATOMS, NUM_FEATURES), jnp.float32),
    scratch_types=[
        pltpu.VMEM((_JPW, _CHUNK), jnp.int32),
    ] + [pltpu.VMEM((_CHUNK, NUM_FEATURES), jnp.float32)] * _NBUF
      + [pltpu.SemaphoreType.DMA] * (2 * _NBUF),
)
def _sc_gather(table_hbm, z3d_hbm, out_hbm, idx_all,
               buf0, buf1, buf2, buf3,
               g0, g1, g2, g3, s0, s1, s2, s3):
    wid = lax.axis_index("s") * _NC + lax.axis_index("c")
    bufs = [buf0, buf1, buf2, buf3]
    gsems = [g0, g1, g2, g3]
    ssems = [s0, s1, s2, s3]

    def gather_desc(j, b):
        return pltpu.make_async_copy(
            table_hbm.at[idx_all.at[j]], bufs[b], gsems[b])

    def store_desc(j, b):
        row0 = (j * _NW + wid) * _CHUNK
        return pltpu.make_async_copy(
            bufs[b], out_hbm.at[pl.ds(row0, _CHUNK)], ssems[b])

    pltpu.sync_copy(z3d_hbm.at[wid], idx_all)
    gather_desc(0, 0).start()
    gather_desc(1, 1).start()

    gather_desc(2, 2).start()
    gather_desc(0, 0).wait()
    store_desc(0, 0).start()
    gather_desc(3, 3).start()
    gather_desc(1, 1).wait()
    store_desc(1, 1).start()

    @pl.loop(0, 5)
    def _steady(gi):
        for b in range(_NBUF):
            j = 2 + gi * _NBUF + b
            store_desc(j - 2, b).wait()
            gather_desc(j + 2, b).start()
            jb = (2 + b) % _NBUF
            gather_desc(j, jb).wait()
            store_desc(j, jb).start()

    store_desc(20, 0).wait()
    gather_desc(22, 2).wait()
    store_desc(22, 2).start()
    store_desc(21, 1).wait()
    gather_desc(23, 3).wait()
    store_desc(23, 3).start()
    store_desc(22, 2).wait()
    store_desc(23, 3).wait()

    c_tail = 768 + wid

    @pl.when(c_tail <= _N_FULL)
    def _():
        pltpu.async_copy(table_hbm.at[idx_all.at[_JPW - 1]], buf0, g0).wait()

        @pl.when(c_tail < _N_FULL)
        def _():
            pltpu.sync_copy(buf0, out_hbm.at[pl.ds(c_tail * _CHUNK, _CHUNK)])

        @pl.when(c_tail == _N_FULL)
        def _():
            pltpu.sync_copy(
                buf0.at[pl.ds(0, _REM)],
                out_hbm.at[pl.ds(_N_FULL * _CHUNK, _REM)])


def kernel(z, elec, W_elec, nuclare_table, W_ls, b_ls):
    table = _compute_table(elec, W_elec, nuclare_table, W_ls, b_ls)
    z_pad = jnp.pad(z, (0, _NCHUNKS * _CHUNK - N_ATOMS))
    z3d = z_pad.reshape(_JPW, _NW, _CHUNK).transpose(1, 0, 2)
    return _sc_gather(table, z3d)


# speedup vs baseline: 9.5880x; 4.3720x over previous
import functools

import jax
import jax.numpy as jnp
from jax import lax
from jax.experimental import pallas as pl
from jax.experimental.pallas import tpu as pltpu
from jax.experimental.pallas import tpu_sc as plsc

NUM_FEATURES = 128
VOCAB = 37
N_ATOMS = 100000

_NC = 2
_NS = 16
_NW = _NC * _NS

_CHUNK = 128
_NCHUNKS = 800
_N_FULL = N_ATOMS // _CHUNK
_REM = N_ATOMS - _N_FULL * _CHUNK
_JPW = _NCHUNKS // _NW
_NBUF = 4


def _table_body(elec_ref, we_ref, nuc_ref, wls_ref, b_ref, out_ref):
    h = nuc_ref[...] + lax.dot_general(
        elec_ref[...], we_ref[...], (((1,), (1,)), ((), ())),
        preferred_element_type=jnp.float32)
    o = lax.dot_general(
        h, wls_ref[...], (((1,), (1,)), ((), ())),
        preferred_element_type=jnp.float32) + b_ref[...]
    out_ref[...] = o * jax.nn.sigmoid(o)


def _compute_table(elec, W_elec, nuclare_table, W_ls, b_ls):
    return pl.pallas_call(
        _table_body,
        out_shape=jax.ShapeDtypeStruct((VOCAB, NUM_FEATURES), jnp.float32),
    )(elec, W_elec, nuclare_table, W_ls, b_ls.reshape(1, NUM_FEATURES))


_mesh = plsc.VectorSubcoreMesh(core_axis_name="c", subcore_axis_name="s")


@functools.partial(
    pl.kernel,
    mesh=_mesh,
    out_type=jax.ShapeDtypeStruct((N_ATOMS, NUM_FEATURES), jnp.float32),
    scratch_types=[
        pltpu.VMEM((_JPW, _CHUNK), jnp.int32),
        pltpu.VMEM_SHARED((VOCAB, NUM_FEATURES), jnp.float32),
    ] + [pltpu.VMEM((_CHUNK, NUM_FEATURES), jnp.float32)] * _NBUF
      + [pltpu.SemaphoreType.DMA] * (2 * _NBUF),
)
def _sc_gather(table_hbm, z3d_hbm, out_hbm, idx_all, table_sp,
               buf0, buf1, buf2, buf3,
               g0, g1, g2, g3, s0, s1, s2, s3):
    wid = lax.axis_index("s") * _NC + lax.axis_index("c")
    bufs = [buf0, buf1, buf2, buf3]
    gsems = [g0, g1, g2, g3]
    ssems = [s0, s1, s2, s3]

    @pl.when(lax.axis_index("s") == 0)
    def _():
        pltpu.sync_copy(table_hbm, table_sp)

    plsc.subcore_barrier()

    def gather_desc(j, b):
        return pltpu.make_async_copy(
            table_sp.at[idx_all.at[j]], bufs[b], gsems[b])

    def store_desc(j, b):
        row0 = (j * _NW + wid) * _CHUNK
        return pltpu.make_async_copy(
            bufs[b], out_hbm.at[pl.ds(row0, _CHUNK)], ssems[b])

    pltpu.sync_copy(z3d_hbm.at[wid], idx_all)
    gather_desc(0, 0).start()
    gather_desc(1, 1).start()

    gather_desc(2, 2).start()
    gather_desc(0, 0).wait()
    store_desc(0, 0).start()
    gather_desc(3, 3).start()
    gather_desc(1, 1).wait()
    store_desc(1, 1).start()

    @pl.loop(0, 5)
    def _steady(gi):
        for b in range(_NBUF):
            j = 2 + gi * _NBUF + b
            store_desc(j - 2, b).wait()
            gather_desc(j + 2, b).start()
            jb = (2 + b) % _NBUF
            gather_desc(j, jb).wait()
            store_desc(j, jb).start()

    store_desc(20, 0).wait()
    gather_desc(22, 2).wait()
    store_desc(22, 2).start()
    store_desc(21, 1).wait()
    gather_desc(23, 3).wait()
    store_desc(23, 3).start()
    store_desc(22, 2).wait()
    store_desc(23, 3).wait()

    c_tail = 768 + wid

    @pl.when(c_tail <= _N_FULL)
    def _():
        pltpu.async_copy(table_sp.at[idx_all.at[_JPW - 1]], buf0, g0).wait()

        @pl.when(c_tail < _N_FULL)
        def _():
            pltpu.sync_copy(buf0, out_hbm.at[pl.ds(c_tail * _CHUNK, _CHUNK)])

        @pl.when(c_tail == _N_FULL)
        def _():
            pltpu.sync_copy(
                buf0.at[pl.ds(0, _REM)],
                out_hbm.at[pl.ds(_N_FULL * _CHUNK, _REM)])


def kernel(z, elec, W_elec, nuclare_table, W_ls, b_ls):
    table = _compute_table(elec, W_elec, nuclare_table, W_ls, b_ls)
    z_pad = jnp.pad(z, (0, _NCHUNKS * _CHUNK - N_ATOMS))
    z3d = z_pad.reshape(_JPW, _NW, _CHUNK).transpose(1, 0, 2)
    return _sc_gather(table, z3d)


# speedup vs baseline: 9.6938x; 1.0110x over previous
import functools

import jax
import jax.numpy as jnp
from jax import lax
from jax.experimental import pallas as pl
from jax.experimental.pallas import tpu as pltpu
from jax.experimental.pallas import tpu_sc as plsc

NUM_FEATURES = 128
VOCAB = 37
N_ATOMS = 100000

_NC = 2
_NS = 16
_NW = _NC * _NS

_CHUNK = 128
_NCHUNKS = 800
_N_FULL = N_ATOMS // _CHUNK
_REM = N_ATOMS - _N_FULL * _CHUNK
_JPW = _NCHUNKS // _NW
_NBUF = 6
_LOOK = _NBUF // 2


def _table_body(elec_ref, we_ref, nuc_ref, wls_ref, b_ref, out_ref):
    h = nuc_ref[...] + lax.dot_general(
        elec_ref[...], we_ref[...], (((1,), (1,)), ((), ())),
        preferred_element_type=jnp.float32)
    o = lax.dot_general(
        h, wls_ref[...], (((1,), (1,)), ((), ())),
        preferred_element_type=jnp.float32) + b_ref[...]
    out_ref[...] = o * jax.nn.sigmoid(o)


def _compute_table(elec, W_elec, nuclare_table, W_ls, b_ls):
    return pl.pallas_call(
        _table_body,
        out_shape=jax.ShapeDtypeStruct((VOCAB, NUM_FEATURES), jnp.float32),
    )(elec, W_elec, nuclare_table, W_ls, b_ls.reshape(1, NUM_FEATURES))


_mesh = plsc.VectorSubcoreMesh(core_axis_name="c", subcore_axis_name="s")


@functools.partial(
    pl.kernel,
    mesh=_mesh,
    out_type=jax.ShapeDtypeStruct((N_ATOMS, NUM_FEATURES), jnp.float32),
    scratch_types=[
        pltpu.VMEM((_JPW, _CHUNK), jnp.int32),
        pltpu.VMEM_SHARED((VOCAB, NUM_FEATURES), jnp.float32),
    ] + [pltpu.VMEM((_CHUNK, NUM_FEATURES), jnp.float32)] * _NBUF
      + [pltpu.SemaphoreType.DMA] * (2 * _NBUF),
)
def _sc_gather(table_hbm, z3d_hbm, out_hbm, idx_all, table_sp,
               buf0, buf1, buf2, buf3, buf4, buf5,
               g0, g1, g2, g3, g4, g5, s0, s1, s2, s3, s4, s5):
    wid = lax.axis_index("s") * _NC + lax.axis_index("c")
    bufs = [buf0, buf1, buf2, buf3, buf4, buf5]
    gsems = [g0, g1, g2, g3, g4, g5]
    ssems = [s0, s1, s2, s3, s4, s5]

    @pl.when(lax.axis_index("s") == 0)
    def _():
        pltpu.sync_copy(table_hbm, table_sp)

    plsc.subcore_barrier()

    def gather_desc(j, b):
        return pltpu.make_async_copy(
            table_sp.at[idx_all.at[j]], bufs[b], gsems[b])

    def store_desc(j, b):
        row0 = (j * _NW + wid) * _CHUNK
        return pltpu.make_async_copy(
            bufs[b], out_hbm.at[pl.ds(row0, _CHUNK)], ssems[b])

    pltpu.sync_copy(z3d_hbm.at[wid], idx_all)
    for j in range(_LOOK):
        gather_desc(j, j).start()

    for j in range(_LOOK):
        gather_desc(j + _LOOK, j + _LOOK).start()
        gather_desc(j, j).wait()
        store_desc(j, j).start()

    @pl.loop(0, 3)
    def _steady(gi):
        for b in range(_NBUF):
            j = _LOOK + gi * _NBUF + b
            store_desc(j - _LOOK, b).wait()
            gather_desc(j + _LOOK, b).start()
            jb = (_LOOK + b) % _NBUF
            gather_desc(j, jb).wait()
            store_desc(j, jb).start()

    for j in range(21, 24):
        store_desc(j - _LOOK, (j - _LOOK) % _NBUF).wait()
        if j + _LOOK <= _JPW - 1:
            gather_desc(j + _LOOK, (j + _LOOK) % _NBUF).start()
        gather_desc(j, j % _NBUF).wait()
        store_desc(j, j % _NBUF).start()
    for j in range(21, 24):
        store_desc(j, j % _NBUF).wait()

    gather_desc(_JPW - 1, (_JPW - 1) % _NBUF).wait()
    tail_buf = bufs[(_JPW - 1) % _NBUF]
    c_tail = (_JPW - 1) * _NW + wid

    @pl.when(c_tail < _N_FULL)
    def _():
        pltpu.sync_copy(tail_buf, out_hbm.at[pl.ds(c_tail * _CHUNK, _CHUNK)])

    @pl.when(c_tail == _N_FULL)
    def _():
        pltpu.sync_copy(
            tail_buf.at[pl.ds(0, _REM)],
            out_hbm.at[pl.ds(_N_FULL * _CHUNK, _REM)])


def kernel(z, elec, W_elec, nuclare_table, W_ls, b_ls):
    table = _compute_table(elec, W_elec, nuclare_table, W_ls, b_ls)
    z_pad = jnp.pad(z, (0, _NCHUNKS * _CHUNK - N_ATOMS))
    z3d = z_pad.reshape(_JPW, _NW, _CHUNK).transpose(1, 0, 2)
    return _sc_gather(table, z3d)
